# baseline (speedup 1.0000x reference)
import functools

import numpy as np
import jax
import jax.numpy as jnp
from jax import lax
from jax.experimental import pallas as pl
from jax.experimental.pallas import tpu as pltpu
from jax.experimental.pallas import tpu_sc as plsc

B, S, E, V = 4096, 200, 64, 100000
GROUP = 5
AVOCAB = 52
NC, NS, L = 2, 16, 16
NW = NC * NS
BPW = B // NW
SPAD = 208
NJ = E // L
NSLOT = 4

_PERM = np.concatenate([np.arange(0, 32, 2), np.arange(1, 32, 2),
                        np.arange(32, 64, 2), np.arange(33, 64, 2)])


def _rsqrt16(v):
  i = plsc.bitcast(v, jnp.int32)
  y = plsc.bitcast(
      jnp.full((L,), 0x5F3759DF, jnp.int32) - lax.shift_right_logical(i, 1),
      jnp.float32)
  h = v * 0.5
  for _ in range(3):
    y = y * (1.5 - h * y * y)
  return y


def _body(ids_hbm, word_hbm, pos_hbm, anum_hbm, asset_hbm, g_hbm, be_hbm,
          out_hbm, posasset, asset, anum, gam, bet, ids_all, rows4, outb2,
          sg0, sg1, sg2, sg3, so0, so1):
  wid = lax.axis_index("s") * NC + lax.axis_index("c")
  base = wid * BPW
  sem_g = (sg0, sg1, sg2, sg3)
  sem_o = (so0, so1)

  def gather_copy(r, c):
    return pltpu.make_async_copy(
        word_hbm.at[ids_all.at[r, pl.ds(0, S)]], rows4.at[c], sem_g[c])

  def out_copy(r, co):
    return pltpu.make_async_copy(outb2.at[co], out_hbm.at[base + r], sem_o[co])

  @pl.loop(0, BPW)
  def _(rr):
    ids_all[rr, pl.ds(SPAD - L, L)] = jnp.zeros((L,), jnp.int32)
  pltpu.sync_copy(ids_hbm.at[pl.ds(base, BPW)], ids_all.at[:, pl.ds(0, S)])
  pltpu.sync_copy(pos_hbm.at[pl.ds(0, S)], posasset)
  pltpu.sync_copy(asset_hbm, asset)
  pltpu.sync_copy(anum_hbm, anum)
  pltpu.sync_copy(g_hbm, gam)
  pltpu.sync_copy(be_hbm, bet)

  @pl.loop(0, S)
  def _(t):
    a = t // GROUP
    for j in range(NJ):
      sl = pl.ds(j * L, L)
      posasset[t, sl] = posasset[t, sl] + asset[a, sl]

  gvec = [gam[pl.ds(j * L, L)] for j in range(NJ)]
  bvec = [bet[pl.ds(j * L, L)] for j in range(NJ)]
  iota2 = lax.iota(jnp.int32, L) * 2
  offs = [iota2, iota2 + 1, iota2 + 32, iota2 + 33]

  for c in range(NSLOT - 1):
    gather_copy(c, c).start()

  @pl.loop(0, BPW // NSLOT)
  def _(h):
    for c in range(NSLOT):
      r = NSLOT * h + c
      f = (c + NSLOT - 1) % NSLOT
      co = c % 2
      cco = jnp.full((L,), co, jnp.int32)

      @pl.when(r + NSLOT - 1 < BPW)
      def _():
        gather_copy(r + NSLOT - 1, f).start()

      gather_copy(r, c).wait()

      cnt = jnp.zeros((L,), jnp.int32)
      one = jnp.ones((L,), jnp.int32)
      zero = jnp.zeros((L,), jnp.int32)
      for k in range(SPAD // L):
        cnt = cnt + jnp.where(ids_all[r, pl.ds(k * L, L)] != 0, one, zero)
      aidx = jnp.sum(cnt) // GROUP
      avec = [anum[aidx, pl.ds(j * L, L)] for j in range(NJ)]

      @pl.when(r >= 2)
      def _():
        out_copy(r - 2, co).wait()

      @plsc.parallel_loop(0, S, unroll=2)
      def _(t):
        w01 = plsc.unpack(rows4[c, t, pl.ds(0, 2 * L)],
                          format=plsc.PackFormat.INTERLEAVED)
        w23 = plsc.unpack(rows4[c, t, pl.ds(2 * L, 2 * L)],
                          format=plsc.PackFormat.INTERLEAVED)
        w = [w01[0], w01[1], w23[0], w23[1]]
        x = [w[j] + posasset[t, pl.ds(j * L, L)] + avec[j] for j in range(NJ)]
        sv = (x[0] + x[1]) + (x[2] + x[3])
        tot = jnp.sum(sv)
        q = [xj * xj for xj in x]
        qv = (q[0] + q[1]) + (q[2] + q[3])
        tot2 = jnp.sum(qv)
        mean = tot * (1.0 / E)
        var = tot2 * (1.0 / E) - mean * mean
        inv = _rsqrt16(jnp.broadcast_to(var + 1e-5, (L,)))
        tt = jnp.full((L,), t, jnp.int32)
        for j in range(NJ):
          y = (x[j] - mean) * inv * gvec[j] + bvec[j]
          plsc.store_scatter(outb2, [cco, tt, offs[j]], y)

      out_copy(r, co).start()

  out_copy(BPW - 2, 0).wait()
  out_copy(BPW - 1, 1).wait()


_mesh = plsc.VectorSubcoreMesh(
    core_axis_name="c", subcore_axis_name="s", num_cores=NC, num_subcores=NS)

_kern = functools.partial(
    pl.kernel,
    out_type=jax.ShapeDtypeStruct((B, S, E), jnp.float32),
    mesh=_mesh,
    compiler_params=pltpu.CompilerParams(
        needs_layout_passes=False, use_tc_tiling_on_sc=False),
    scratch_types=[
        pltpu.VMEM((S, E), jnp.float32),
        pltpu.VMEM((AVOCAB, E), jnp.float32),
        pltpu.VMEM((AVOCAB, E), jnp.float32),
        pltpu.VMEM((E,), jnp.float32),
        pltpu.VMEM((E,), jnp.float32),
        pltpu.VMEM((BPW, SPAD), jnp.int32),
        pltpu.VMEM((NSLOT, S, E), jnp.bfloat16),
        pltpu.VMEM((2, S, E), jnp.float32),
        pltpu.SemaphoreType.DMA,
        pltpu.SemaphoreType.DMA,
        pltpu.SemaphoreType.DMA,
        pltpu.SemaphoreType.DMA,
        pltpu.SemaphoreType.DMA,
        pltpu.SemaphoreType.DMA,
    ],
)(_body)


@jax.jit
def kernel(input_ids, word_table, pos_table, asset_num_table, asset_table,
           attr_table, ln_gamma, ln_beta):
  del attr_table
  ids = input_ids.astype(jnp.int32)
  word_bf = word_table.astype(jnp.bfloat16)
  perm = jnp.asarray(_PERM)
  return _kern(ids, word_bf, pos_table[:, perm], asset_num_table[:, perm],
               asset_table[:, perm], ln_gamma[perm], ln_beta[perm])


# speedup vs baseline: 1.0690x; 1.0690x over previous
import functools

import jax
import jax.numpy as jnp
from jax import lax
from jax.experimental import pallas as pl
from jax.experimental.pallas import tpu as pltpu
from jax.experimental.pallas import tpu_sc as plsc

B, S, E, V = 4096, 200, 64, 100000
GROUP = 5
AVOCAB = 52
NC, NS, L = 2, 16, 16
NW = NC * NS
BPW = B // NW
SPAD = 208
NJ = E // L
NSLOT = 4


def _rsqrt16(v):
  i = plsc.bitcast(v, jnp.int32)
  y = plsc.bitcast(
      jnp.full((L,), 0x5F3759DF, jnp.int32) - lax.shift_right_logical(i, 1),
      jnp.float32)
  h = v * 0.5
  for _ in range(3):
    y = y * (1.5 - h * y * y)
  return y


def _body(ids_hbm, word_hbm, pos_hbm, anum_hbm, asset_hbm, g_hbm, be_hbm,
          out_hbm, posasset, asset, anum, gam, bet, ids_all, rows4, outb2,
          sg0, sg1, sg2, sg3, so0, so1):
  wid = lax.axis_index("s") * NC + lax.axis_index("c")
  base = wid * BPW
  sem_g = (sg0, sg1, sg2, sg3)
  sem_o = (so0, so1)

  def gather_copy(r, c):
    return pltpu.make_async_copy(
        word_hbm.at[ids_all.at[r, pl.ds(0, S)]], rows4.at[c], sem_g[c])

  def out_copy(r, co):
    return pltpu.make_async_copy(outb2.at[co], out_hbm.at[base + r], sem_o[co])

  @pl.loop(0, BPW)
  def _(rr):
    ids_all[rr, pl.ds(SPAD - L, L)] = jnp.zeros((L,), jnp.int32)
  pltpu.sync_copy(ids_hbm.at[pl.ds(base, BPW)], ids_all.at[:, pl.ds(0, S)])
  pltpu.sync_copy(pos_hbm.at[pl.ds(0, S)], posasset)
  pltpu.sync_copy(asset_hbm, asset)
  pltpu.sync_copy(anum_hbm, anum)
  pltpu.sync_copy(g_hbm, gam)
  pltpu.sync_copy(be_hbm, bet)

  @pl.loop(0, S)
  def _(t):
    a = t // GROUP
    for j in range(NJ):
      sl = pl.ds(j * L, L)
      posasset[t, sl] = posasset[t, sl] + asset[a, sl]

  gvec = [gam[pl.ds(j * L, L)] for j in range(NJ)]
  bvec = [bet[pl.ds(j * L, L)] for j in range(NJ)]

  for c in range(NSLOT - 1):
    gather_copy(c, c).start()

  @pl.loop(0, BPW // NSLOT)
  def _(h):
    for c in range(NSLOT):
      r = NSLOT * h + c
      f = (c + NSLOT - 1) % NSLOT
      co = c % 2

      @pl.when(r + NSLOT - 1 < BPW)
      def _():
        gather_copy(r + NSLOT - 1, f).start()

      gather_copy(r, c).wait()

      cnt = jnp.zeros((L,), jnp.int32)
      one = jnp.ones((L,), jnp.int32)
      zero = jnp.zeros((L,), jnp.int32)
      for k in range(SPAD // L):
        cnt = cnt + jnp.where(ids_all[r, pl.ds(k * L, L)] != 0, one, zero)
      aidx = jnp.sum(cnt) // GROUP
      avec = [anum[aidx, pl.ds(j * L, L)] for j in range(NJ)]

      @pl.when(r >= 2)
      def _():
        out_copy(r - 2, co).wait()

      @plsc.parallel_loop(0, S, unroll=2)
      def _(t):
        x = [rows4[c, t, pl.ds(j * L, L)] + posasset[t, pl.ds(j * L, L)]
             + avec[j] for j in range(NJ)]
        sv = (x[0] + x[1]) + (x[2] + x[3])
        tot = jnp.sum(sv)
        q = [xj * xj for xj in x]
        qv = (q[0] + q[1]) + (q[2] + q[3])
        tot2 = jnp.sum(qv)
        mean = tot * (1.0 / E)
        var = tot2 * (1.0 / E) - mean * mean
        inv = _rsqrt16(jnp.broadcast_to(var + 1e-5, (L,)))
        for j in range(NJ):
          outb2[co, t, pl.ds(j * L, L)] = (x[j] - mean) * inv * gvec[j] + bvec[j]

      out_copy(r, co).start()

  out_copy(BPW - 2, 0).wait()
  out_copy(BPW - 1, 1).wait()


_mesh = plsc.VectorSubcoreMesh(
    core_axis_name="c", subcore_axis_name="s", num_cores=NC, num_subcores=NS)

_kern = functools.partial(
    pl.kernel,
    out_type=jax.ShapeDtypeStruct((B, S, E), jnp.float32),
    mesh=_mesh,
    compiler_params=pltpu.CompilerParams(
        needs_layout_passes=False, use_tc_tiling_on_sc=False),
    scratch_types=[
        pltpu.VMEM((S, E), jnp.float32),
        pltpu.VMEM((AVOCAB, E), jnp.float32),
        pltpu.VMEM((AVOCAB, E), jnp.float32),
        pltpu.VMEM((E,), jnp.float32),
        pltpu.VMEM((E,), jnp.float32),
        pltpu.VMEM((BPW, SPAD), jnp.int32),
        pltpu.VMEM((NSLOT, S, E), jnp.float32),
        pltpu.VMEM((2, S, E), jnp.float32),
        pltpu.SemaphoreType.DMA,
        pltpu.SemaphoreType.DMA,
        pltpu.SemaphoreType.DMA,
        pltpu.SemaphoreType.DMA,
        pltpu.SemaphoreType.DMA,
        pltpu.SemaphoreType.DMA,
    ],
)(_body)


@jax.jit
def kernel(input_ids, word_table, pos_table, asset_num_table, asset_table,
           attr_table, ln_gamma, ln_beta):
  del attr_table
  ids = input_ids.astype(jnp.int32)
  return _kern(ids, word_table, pos_table, asset_num_table, asset_table,
               ln_gamma, ln_beta)
